# baseline (speedup 1.0000x reference)
import functools

import jax
import jax.numpy as jnp
from jax import lax
from jax.experimental import pallas as pl
from jax.experimental.pallas import tpu as pltpu
from jax.experimental.pallas import tpu_sc as plsc

_VOCAB = 16
_HIDDEN = 128
_IDX_CHUNK = 128


@functools.cache
def _make_kernel(n_tokens: int):
    info = plsc.get_sparse_core_info()
    nc, ns = info.num_cores, info.num_subcores
    nw = nc * ns
    tok_per_w = n_tokens // nw
    assert tok_per_w * nw == n_tokens and tok_per_w % _IDX_CHUNK == 0
    assert ns == _VOCAB
    chunks = tok_per_w // _IDX_CHUNK
    mesh = plsc.VectorSubcoreMesh(core_axis_name="c", subcore_axis_name="s")

    @functools.partial(
        pl.kernel,
        mesh=mesh,
        compiler_params=pltpu.CompilerParams(use_tc_tiling_on_sc=False),
        out_type=jax.ShapeDtypeStruct(
            (n_tokens // _IDX_CHUNK, _IDX_CHUNK, _VOCAB), jnp.float32),
        scratch_types=[
            pltpu.VMEM((chunks, _IDX_CHUNK), jnp.int32),
            pltpu.VMEM((chunks, _IDX_CHUNK, _VOCAB), jnp.float32),
            pltpu.VMEM((_HIDDEN,), jnp.float32),
            pltpu.VMEM((_HIDDEN, _VOCAB), jnp.float32),
            pltpu.VMEM((_VOCAB,), jnp.float32),
            pltpu.VMEM_SHARED((_VOCAB, _VOCAB), jnp.float32),
            pltpu.SemaphoreType.DMA,
            pltpu.SemaphoreType.DMA,
        ],
    )
    def tiny_lm_k(e_hbm, wt_hbm, idx_hbm, out_hbm,
                  idx_v, rows_v, e_row, wt_v, m_row, m_sp, sem_idx, sem_g):
        s = lax.axis_index("s")
        wid = s * nc + lax.axis_index("c")
        row0 = wid * chunks
        idx_cp = pltpu.async_copy(
            idx_hbm.at[pl.ds(row0, chunks)], idx_v, sem_idx)
        pltpu.sync_copy(e_hbm.at[s], e_row)
        pltpu.sync_copy(wt_hbm, wt_v)
        acc = jnp.zeros((_VOCAB,), jnp.float32)
        for k in range(_HIDDEN // 16):
            ev = e_row[pl.ds(16 * k, 16)]
            for l in range(16):
                acc = acc + ev[l] * wt_v[16 * k + l]
        m_row[...] = acc
        pltpu.sync_copy(m_row, m_sp.at[s])
        plsc.subcore_barrier()
        idx_cp.wait()
        copies = [
            pltpu.async_copy(m_sp.at[idx_v.at[j]], rows_v.at[j], sem_g)
            for j in range(chunks)
        ]
        for c in copies:
            c.wait()
        pltpu.sync_copy(rows_v, out_hbm.at[pl.ds(row0, chunks)])

    return tiny_lm_k


def kernel(input_ids, embed_table, proj_w):
    b, s = input_ids.shape
    n_tokens = b * s
    ids = input_ids.reshape(n_tokens // _IDX_CHUNK, _IDX_CHUNK)
    ids = ids.astype(jnp.int32)
    out = _make_kernel(n_tokens)(embed_table, proj_w.T, ids)
    return out.reshape(b, s, _VOCAB)
